# baseline (speedup 1.0000x reference)
import functools

import jax
import jax.numpy as jnp
from jax import lax
from jax.experimental import pallas as pl
from jax.experimental.pallas import tpu as pltpu
from jax.experimental.pallas import tpu_sc as plsc

B = 16384
L = 20
D = 64
LANES = 16
VOCAB = 1000000

NC = 2
NS = 16
NW = NC * NS

BPW = B // NW
CH = 16
NCHUNK = BPW // CH
ROWS = CH * L
GSZ = 80
NG = ROWS // GSZ
DP = 2 * D
LPAD = 32


def _body(idx_hbm, off_hbm, table_hbm, out_hbm,
          idx_v, off_v, rows_v, out_v, sem0, sem1):
    wid = lax.axis_index("s") * NC + lax.axis_index("c")
    base = wid * BPW
    sems = (sem0, sem1)

    pltpu.sync_copy(idx_hbm.at[pl.ds(base * L, BPW * L)], idx_v)
    pltpu.sync_copy(off_hbm.at[pl.ds(base * LPAD, BPW * LPAD)], off_v)

    def fire(g, slot):
        for j in range(NG):
            pltpu.async_copy(
                table_hbm.at[idx_v.at[pl.ds(g * ROWS + j * GSZ, GSZ)]],
                rows_v.at[slot, pl.ds(j * GSZ, GSZ)],
                sems[slot],
            )

    def drain(slot):
        pltpu.make_async_copy(
            table_hbm.at[pl.ds(0, ROWS)], rows_v.at[slot], sems[slot]
        ).wait()

    def reduce_store(g, slot):
        @plsc.parallel_loop(0, CH, 1, unroll=2)
        def _red(c):
            off_a = off_v[pl.ds((g * CH + c) * LPAD, LANES)]
            off_b = off_v[pl.ds((g * CH + c) * LPAD + LANES, LANES)]
            accs = [jnp.zeros((LANES,), jnp.float32) for _ in range(D // LANES)]
            for t in range(L):
                r = c * L + t
                off = off_a[t] if t < LANES else off_b[t - LANES]
                for dd in range(D // LANES):
                    accs[dd] = accs[dd] + rows_v[slot, r, pl.ds(off + dd * LANES, LANES)]
            for dd in range(D // LANES):
                out_v[slot, pl.ds(c * D + dd * LANES, LANES)] = accs[dd] * (1.0 / L)

        pltpu.sync_copy(
            out_v.at[slot], out_hbm.at[pl.ds((base + g * CH) * D, CH * D)]
        )

    fire(0, 0)

    def step(k, carry):
        g = 2 * k
        fire(g + 1, 1)
        drain(0)
        reduce_store(g, 0)

        @pl.when(k < NCHUNK // 2 - 1)
        def _():
            fire(g + 2, 0)

        drain(1)
        reduce_store(g + 1, 1)
        return carry

    lax.fori_loop(0, NCHUNK // 2, step, 0)


@jax.jit
def _run(idx2, off64, table_pairs):
    mesh = plsc.VectorSubcoreMesh(core_axis_name="c", subcore_axis_name="s")
    f = pl.kernel(
        _body,
        out_type=jax.ShapeDtypeStruct((B * D,), jnp.float32),
        mesh=mesh,
        scratch_types=[
            pltpu.VMEM((BPW * L,), jnp.int32),
            pltpu.VMEM((BPW * LPAD,), jnp.int32),
            pltpu.VMEM((2, ROWS, DP), jnp.float32),
            pltpu.VMEM((2, CH * D), jnp.float32),
            pltpu.SemaphoreType.DMA,
            pltpu.SemaphoreType.DMA,
        ],
    )
    return f(idx2, off64, table_pairs).reshape(B, D)


def kernel(indices, table):
    idx = indices.astype(jnp.int32).reshape(B * L)
    idx2 = idx >> 1
    off64 = jnp.pad(((indices.astype(jnp.int32) & 1) << 6), ((0, 0), (0, LPAD - L))).reshape(B * LPAD)
    table_pairs = table.astype(jnp.float32).reshape(VOCAB // 2, 2 * D)
    return _run(idx2, off64, table_pairs)


# speedup vs baseline: 1.5241x; 1.5241x over previous
import functools

import jax
import jax.numpy as jnp
from jax import lax
from jax.experimental import pallas as pl
from jax.experimental.pallas import tpu as pltpu
from jax.experimental.pallas import tpu_sc as plsc

B = 16384
L = 20
D = 64
LANES = 16
VOCAB = 1000000

NC = 2
NS = 16
NW = NC * NS

BPW = B // NW
CH = 16
NCHUNK = BPW // CH
ROWS = CH * L
GSZ = 80
NG = ROWS // GSZ
DP = 2 * D
LPAD = 32


def _body(idx_hbm, off_hbm, table_hbm, out_hbm,
          idx_v, off_v, rows_v, out_v, sem0, sem1):
    wid = lax.axis_index("s") * NC + lax.axis_index("c")
    base = wid * BPW
    sems = (sem0, sem1)

    pltpu.sync_copy(idx_hbm.at[pl.ds(base * L, BPW * L)], idx_v)
    pltpu.sync_copy(off_hbm.at[pl.ds(base * LPAD, BPW * LPAD)], off_v)

    def fire(g, slot):
        for j in range(NG):
            pltpu.async_copy(
                table_hbm.at[idx_v.at[pl.ds(g * ROWS + j * GSZ, GSZ)]],
                rows_v.at[slot, pl.ds(j * GSZ, GSZ)],
                sems[slot],
            )

    def drain(slot):
        pltpu.make_async_copy(
            table_hbm.at[pl.ds(0, ROWS)], rows_v.at[slot], sems[slot]
        ).wait()

    def reduce_store(g, slot):
        @plsc.parallel_loop(0, CH, 1, unroll=2)
        def _red(c):
            off_a = off_v[pl.ds((g * CH + c) * LPAD, LANES)]
            off_b = off_v[pl.ds((g * CH + c) * LPAD + LANES, LANES)]
            accs = [jnp.zeros((LANES,), jnp.float32) for _ in range(D // LANES)]
            for t in range(L):
                r = c * L + t
                off = off_a[t] if t < LANES else off_b[t - LANES]
                for dd in range(D // LANES):
                    accs[dd] = accs[dd] + rows_v[slot, r, pl.ds(off + dd * LANES, LANES)]
            for dd in range(D // LANES):
                out_v[slot, pl.ds(c * D + dd * LANES, LANES)] = accs[dd] * (1.0 / L)

        pltpu.sync_copy(
            out_v.at[slot], out_hbm.at[pl.ds((base + g * CH) * D, CH * D)]
        )

    fire(0, 0)

    def step(k, carry):
        g = 2 * k
        fire(g + 1, 1)
        drain(0)
        reduce_store(g, 0)

        @pl.when(k < NCHUNK // 2 - 1)
        def _():
            fire(g + 2, 0)

        drain(1)
        reduce_store(g + 1, 1)
        return carry

    lax.fori_loop(0, NCHUNK // 2, step, 0)


TBLK = 4096
VROWS = 500096


def _transpose_body(x_ref, o_ref):
    z = jnp.swapaxes(x_ref[...], 0, 1)
    for gp in range(TBLK // 256):
        y = jnp.concatenate(
            [z[gp * 256:gp * 256 + 128], z[gp * 256 + 128:gp * 256 + 256]],
            axis=1,
        )
        o_ref[pl.ds(gp * 128, 128), :] = y


def _relayout(table_t):
    return pl.pallas_call(
        _transpose_body,
        grid=(pl.cdiv(VOCAB, TBLK),),
        in_specs=[pl.BlockSpec((D, TBLK), lambda i: (0, i))],
        out_specs=pl.BlockSpec((TBLK // 2, 2 * D), lambda i: (i, 0)),
        out_shape=jax.ShapeDtypeStruct((VROWS, 2 * D), jnp.float32),
    )(table_t)


@jax.jit
def _run(idx2, off64, table_pairs):
    mesh = plsc.VectorSubcoreMesh(core_axis_name="c", subcore_axis_name="s")
    f = pl.kernel(
        _body,
        out_type=jax.ShapeDtypeStruct((B * D,), jnp.float32),
        mesh=mesh,
        scratch_types=[
            pltpu.VMEM((BPW * L,), jnp.int32),
            pltpu.VMEM((BPW * LPAD,), jnp.int32),
            pltpu.VMEM((2, ROWS, DP), jnp.float32),
            pltpu.VMEM((2, CH * D), jnp.float32),
            pltpu.SemaphoreType.DMA,
            pltpu.SemaphoreType.DMA,
        ],
    )
    return f(idx2, off64, table_pairs).reshape(B, D)


def kernel(indices, table):
    idx = indices.astype(jnp.int32).reshape(B * L)
    idx2 = ((idx >> 8) << 7) | (idx & 127)
    half = (indices.astype(jnp.int32) >> 7) & 1
    off64 = jnp.pad(half << 6, ((0, 0), (0, LPAD - L))).reshape(B * LPAD)
    table_pairs = _relayout(table.astype(jnp.float32).T)
    return _run(idx2, off64, table_pairs)


# speedup vs baseline: 1.7921x; 1.1759x over previous
import functools

import jax
import jax.numpy as jnp
from jax import lax
from jax.experimental import pallas as pl
from jax.experimental.pallas import tpu as pltpu
from jax.experimental.pallas import tpu_sc as plsc

B = 16384
L = 20
D = 64
LANES = 16
VOCAB = 1000000

NC = 2
NS = 16
NW = NC * NS

BPW = B // NW
CH = 32
NCHUNK = BPW // CH
ROWS = CH * L
GSZ = 128
NG = ROWS // GSZ

TBLK = 4096
VROWS = 500096


def _body(idx_hbm, table_hbm, out_hbm, idx_v, rows_v, out_v, sem0, sem1):
    wid = lax.axis_index("s") * NC + lax.axis_index("c")
    base = wid * BPW
    sems = (sem0, sem1)

    pltpu.sync_copy(idx_hbm.at[pl.ds(base * L, BPW * L)], idx_v)

    def fire(g, slot):
        for j in range(NG):
            pltpu.async_copy(
                table_hbm.at[idx_v.at[pl.ds(g * ROWS + j * GSZ, GSZ)]],
                rows_v.at[slot, pl.ds(j * GSZ, GSZ)],
                sems[slot],
            )

    def drain(slot):
        pltpu.make_async_copy(
            table_hbm.at[pl.ds(0, ROWS)], rows_v.at[slot], sems[slot]
        ).wait()

    def reduce_store(g, slot):
        @plsc.parallel_loop(0, CH, 1, unroll=2)
        def _red(c):
            accs = [jnp.zeros((LANES,), jnp.float32) for _ in range(D // LANES)]
            for t in range(L):
                r = c * L + t
                for dd in range(D // LANES):
                    accs[dd] = accs[dd] + rows_v[slot, r, pl.ds(dd * LANES, LANES)]
            for dd in range(D // LANES):
                out_v[slot, pl.ds(c * D + dd * LANES, LANES)] = accs[dd] * (1.0 / L)

        pltpu.sync_copy(
            out_v.at[slot], out_hbm.at[pl.ds((base + g * CH) * D, CH * D)]
        )

    fire(0, 0)

    def step(k, carry):
        g = 2 * k
        fire(g + 1, 1)
        drain(0)
        reduce_store(g, 0)

        @pl.when(k < NCHUNK // 2 - 1)
        def _():
            fire(g + 2, 0)

        drain(1)
        reduce_store(g + 1, 1)
        return carry

    lax.fori_loop(0, NCHUNK // 2, step, 0)


def _transpose_body(x_ref, o_ref):
    z = jnp.swapaxes(x_ref[...], 0, 1)
    for gp in range(TBLK // 256):
        y = jnp.concatenate(
            [z[gp * 256:gp * 256 + 128], z[gp * 256 + 128:gp * 256 + 256]],
            axis=1,
        )
        o_ref[pl.ds(gp * 128, 128), :] = y


def _relayout(table_t):
    return pl.pallas_call(
        _transpose_body,
        grid=(pl.cdiv(VOCAB, TBLK),),
        in_specs=[pl.BlockSpec((D, TBLK), lambda i: (0, i))],
        out_specs=pl.BlockSpec((TBLK // 2, 2 * D), lambda i: (i, 0)),
        out_shape=jax.ShapeDtypeStruct((VROWS, 2 * D), jnp.float32),
    )(table_t)


@jax.jit
def _run(idx_m, table_rows):
    mesh = plsc.VectorSubcoreMesh(core_axis_name="c", subcore_axis_name="s")
    f = pl.kernel(
        _body,
        out_type=jax.ShapeDtypeStruct((B * D,), jnp.float32),
        mesh=mesh,
        scratch_types=[
            pltpu.VMEM((BPW * L,), jnp.int32),
            pltpu.VMEM((2, ROWS, D), jnp.float32),
            pltpu.VMEM((2, CH * D), jnp.float32),
            pltpu.SemaphoreType.DMA,
            pltpu.SemaphoreType.DMA,
        ],
        compiler_params=pltpu.CompilerParams(use_tc_tiling_on_sc=False),
    )
    return f(idx_m, table_rows).reshape(B, D)


def kernel(indices, table):
    idx = indices.astype(jnp.int32).reshape(B * L)
    idx_m = ((((idx >> 8) << 7) | (idx & 127)) << 1) | ((idx >> 7) & 1)
    table_rows = _relayout(table.astype(jnp.float32).T).reshape(2 * VROWS, D)
    return _run(idx_m, table_rows)


# speedup vs baseline: 2.1479x; 1.1985x over previous
import functools

import jax
import jax.numpy as jnp
from jax import lax
from jax.experimental import pallas as pl
from jax.experimental.pallas import tpu as pltpu
from jax.experimental.pallas import tpu_sc as plsc

B = 16384
L = 20
D = 64
LANES = 16
VOCAB = 1000000

NC = 2
NS = 16
NW = NC * NS

BPW = B // NW
CH = 32
NCHUNK = BPW // CH
ROWS = CH * L
GSZ = 128
NG = ROWS // GSZ

TBLK = 8192
VROWS = 500096


def _body(idx_hbm, table_hbm, out_hbm, idx_v, rows_v, out_v, sem0, sem1):
    wid = lax.axis_index("s") * NC + lax.axis_index("c")
    base = wid * BPW
    sems = (sem0, sem1)

    pltpu.sync_copy(idx_hbm.at[pl.ds(base * L, BPW * L)], idx_v)

    def fire(g, slot):
        for j in range(NG):
            pltpu.async_copy(
                table_hbm.at[idx_v.at[pl.ds(g * ROWS + j * GSZ, GSZ)]],
                rows_v.at[slot, pl.ds(j * GSZ, GSZ)],
                sems[slot],
            )

    def drain(slot):
        pltpu.make_async_copy(
            table_hbm.at[pl.ds(0, ROWS)], rows_v.at[slot], sems[slot]
        ).wait()

    def reduce_store(g, slot):
        @plsc.parallel_loop(0, CH, 1, unroll=2)
        def _red(c):
            accs = [jnp.zeros((LANES,), jnp.float32) for _ in range(D // LANES)]
            for t in range(L):
                r = c * L + t
                for dd in range(D // LANES):
                    accs[dd] = accs[dd] + rows_v[slot, r, pl.ds(dd * LANES, LANES)]
            for dd in range(D // LANES):
                out_v[slot, pl.ds(c * D + dd * LANES, LANES)] = accs[dd] * (1.0 / L)

        pltpu.sync_copy(
            out_v.at[slot], out_hbm.at[pl.ds((base + g * CH) * D, CH * D)]
        )

    fire(0, 0)

    def step(k, carry):
        g = 2 * k
        fire(g + 1, 1)
        drain(0)
        reduce_store(g, 0)

        @pl.when(k < NCHUNK // 2 - 1)
        def _():
            fire(g + 2, 0)

        drain(1)
        reduce_store(g + 1, 1)
        return carry

    lax.fori_loop(0, NCHUNK // 2, step, 0)


def _transpose_body(x_ref, o_ref):
    z = jnp.swapaxes(x_ref[...], 0, 1)
    for gp in range(TBLK // 256):
        y = jnp.concatenate(
            [z[gp * 256:gp * 256 + 128], z[gp * 256 + 128:gp * 256 + 256]],
            axis=1,
        )
        o_ref[pl.ds(gp * 128, 128), :] = y


def _relayout(table_t):
    return pl.pallas_call(
        _transpose_body,
        grid=(pl.cdiv(VOCAB, TBLK),),
        in_specs=[pl.BlockSpec((D, TBLK), lambda i: (0, i))],
        out_specs=pl.BlockSpec((TBLK // 2, 2 * D), lambda i: (i, 0)),
        out_shape=jax.ShapeDtypeStruct((VROWS, 2 * D), jnp.float32),
    )(table_t)


@jax.jit
def _run(idx_m, table_rows):
    mesh = plsc.VectorSubcoreMesh(core_axis_name="c", subcore_axis_name="s")
    f = pl.kernel(
        _body,
        out_type=jax.ShapeDtypeStruct((B * D,), jnp.float32),
        mesh=mesh,
        scratch_types=[
            pltpu.VMEM((BPW * L,), jnp.int32),
            pltpu.VMEM((2, ROWS, D), jnp.float32),
            pltpu.VMEM((2, CH * D), jnp.float32),
            pltpu.SemaphoreType.DMA,
            pltpu.SemaphoreType.DMA,
        ],
        compiler_params=pltpu.CompilerParams(use_tc_tiling_on_sc=False),
    )
    return f(idx_m, table_rows).reshape(B, D)


def kernel(indices, table):
    idx = indices.astype(jnp.int32).reshape(B * L)
    idx_m = ((((idx >> 8) << 7) | (idx & 127)) << 1) | ((idx >> 7) & 1)
    table_rows = _relayout(table.astype(jnp.float32).T).reshape(2 * VROWS, D)
    return _run(idx_m, table_rows)


# speedup vs baseline: 2.3775x; 1.1069x over previous
import functools

import jax
import jax.numpy as jnp
from jax import lax
from jax.experimental import pallas as pl
from jax.experimental.pallas import tpu as pltpu
from jax.experimental.pallas import tpu_sc as plsc

B = 16384
L = 20
D = 64
LANES = 16
VOCAB = 1000000

NC = 2
NS = 16
NW = NC * NS

BPW = B // NW
CH = 32
NCHUNK = BPW // CH
ROWS = CH * L
GSZ = 128
NG = ROWS // GSZ

TBLK = 16384
VROWS = 500096


def _body(idx_hbm, table_hbm, out_hbm, idx_v, rows_v, out_v, sem0, sem1):
    wid = lax.axis_index("s") * NC + lax.axis_index("c")
    base = wid * BPW
    sems = (sem0, sem1)

    pltpu.sync_copy(idx_hbm.at[pl.ds(base * L, BPW * L)], idx_v)

    def fire(g, slot):
        for j in range(NG):
            pltpu.async_copy(
                table_hbm.at[idx_v.at[pl.ds(g * ROWS + j * GSZ, GSZ)]],
                rows_v.at[slot, pl.ds(j * GSZ, GSZ)],
                sems[slot],
            )

    def drain(slot):
        pltpu.make_async_copy(
            table_hbm.at[pl.ds(0, ROWS)], rows_v.at[slot], sems[slot]
        ).wait()

    def reduce_store(g, slot):
        @plsc.parallel_loop(0, CH, 1, unroll=2)
        def _red(c):
            accs = [jnp.zeros((LANES,), jnp.float32) for _ in range(D // LANES)]
            for t in range(L):
                r = c * L + t
                for dd in range(D // LANES):
                    accs[dd] = accs[dd] + rows_v[slot, r, pl.ds(dd * LANES, LANES)]
            for dd in range(D // LANES):
                out_v[slot, pl.ds(c * D + dd * LANES, LANES)] = accs[dd] * (1.0 / L)

        pltpu.sync_copy(
            out_v.at[slot], out_hbm.at[pl.ds((base + g * CH) * D, CH * D)]
        )

    fire(0, 0)

    def step(k, carry):
        g = 2 * k
        fire(g + 1, 1)
        drain(0)
        reduce_store(g, 0)

        @pl.when(k < NCHUNK // 2 - 1)
        def _():
            fire(g + 2, 0)

        drain(1)
        reduce_store(g + 1, 1)
        return carry

    lax.fori_loop(0, NCHUNK // 2, step, 0)


def _transpose_body(x_ref, o_ref):
    z = jnp.swapaxes(x_ref[...], 0, 1)
    for gp in range(TBLK // 256):
        y = jnp.concatenate(
            [z[gp * 256:gp * 256 + 128], z[gp * 256 + 128:gp * 256 + 256]],
            axis=1,
        )
        o_ref[pl.ds(gp * 128, 128), :] = y


def _relayout(table_t):
    return pl.pallas_call(
        _transpose_body,
        grid=(pl.cdiv(VOCAB, TBLK),),
        in_specs=[pl.BlockSpec((D, TBLK), lambda i: (0, i))],
        out_specs=pl.BlockSpec((TBLK // 2, 2 * D), lambda i: (i, 0)),
        out_shape=jax.ShapeDtypeStruct((VROWS, 2 * D), jnp.float32),
    )(table_t)


@jax.jit
def _run(idx_m, table_rows):
    mesh = plsc.VectorSubcoreMesh(core_axis_name="c", subcore_axis_name="s")
    f = pl.kernel(
        _body,
        out_type=jax.ShapeDtypeStruct((B * D,), jnp.float32),
        mesh=mesh,
        scratch_types=[
            pltpu.VMEM((BPW * L,), jnp.int32),
            pltpu.VMEM((2, ROWS, D), jnp.float32),
            pltpu.VMEM((2, CH * D), jnp.float32),
            pltpu.SemaphoreType.DMA,
            pltpu.SemaphoreType.DMA,
        ],
        compiler_params=pltpu.CompilerParams(use_tc_tiling_on_sc=False),
    )
    return f(idx_m, table_rows).reshape(B, D)


def kernel(indices, table):
    idx = indices.astype(jnp.int32).reshape(B * L)
    idx_m = ((((idx >> 8) << 7) | (idx & 127)) << 1) | ((idx >> 7) & 1)
    table_rows = _relayout(table.astype(jnp.float32).T).reshape(2 * VROWS, D)
    return _run(idx_m, table_rows)


# speedup vs baseline: 2.5017x; 1.0522x over previous
import functools

import jax
import jax.numpy as jnp
from jax import lax
from jax.experimental import pallas as pl
from jax.experimental.pallas import tpu as pltpu
from jax.experimental.pallas import tpu_sc as plsc

B = 16384
L = 20
D = 64
LANES = 16
VOCAB = 1000000

NC = 2
NS = 16
NW = NC * NS

BPW = B // NW
CH = 32
NCHUNK = BPW // CH
ROWS = CH * L
GSZ = 128
NG = ROWS // GSZ

TBLK = 32768
VROWS = 500096


def _body(idx_hbm, table_hbm, out_hbm, idx_v, rows_v, out_v, sem0, sem1):
    wid = lax.axis_index("s") * NC + lax.axis_index("c")
    base = wid * BPW
    sems = (sem0, sem1)

    pltpu.sync_copy(idx_hbm.at[pl.ds(base * L, BPW * L)], idx_v)

    def fire(g, slot):
        for j in range(NG):
            pltpu.async_copy(
                table_hbm.at[idx_v.at[pl.ds(g * ROWS + j * GSZ, GSZ)]],
                rows_v.at[slot, pl.ds(j * GSZ, GSZ)],
                sems[slot],
            )

    def drain(slot):
        pltpu.make_async_copy(
            table_hbm.at[pl.ds(0, ROWS)], rows_v.at[slot], sems[slot]
        ).wait()

    def reduce_store(g, slot):
        @plsc.parallel_loop(0, CH, 1, unroll=2)
        def _red(c):
            accs = [jnp.zeros((LANES,), jnp.float32) for _ in range(D // LANES)]
            for t in range(L):
                r = c * L + t
                for dd in range(D // LANES):
                    accs[dd] = accs[dd] + rows_v[slot, r, pl.ds(dd * LANES, LANES)]
            for dd in range(D // LANES):
                out_v[slot, pl.ds(c * D + dd * LANES, LANES)] = accs[dd] * (1.0 / L)

        pltpu.sync_copy(
            out_v.at[slot], out_hbm.at[pl.ds((base + g * CH) * D, CH * D)]
        )

    fire(0, 0)

    def step(k, carry):
        g = 2 * k
        fire(g + 1, 1)
        drain(0)
        reduce_store(g, 0)

        @pl.when(k < NCHUNK // 2 - 1)
        def _():
            fire(g + 2, 0)

        drain(1)
        reduce_store(g + 1, 1)
        return carry

    lax.fori_loop(0, NCHUNK // 2, step, 0)


def _transpose_body(x_ref, o_ref):
    z = jnp.swapaxes(x_ref[...], 0, 1)
    for gp in range(TBLK // 256):
        y = jnp.concatenate(
            [z[gp * 256:gp * 256 + 128], z[gp * 256 + 128:gp * 256 + 256]],
            axis=1,
        )
        o_ref[pl.ds(gp * 128, 128), :] = y


def _relayout(table_t):
    return pl.pallas_call(
        _transpose_body,
        grid=(pl.cdiv(VOCAB, TBLK),),
        in_specs=[pl.BlockSpec((D, TBLK), lambda i: (0, i))],
        out_specs=pl.BlockSpec((TBLK // 2, 2 * D), lambda i: (i, 0)),
        out_shape=jax.ShapeDtypeStruct((VROWS, 2 * D), jnp.float32),
    )(table_t)


@jax.jit
def _run(idx_m, table_rows):
    mesh = plsc.VectorSubcoreMesh(core_axis_name="c", subcore_axis_name="s")
    f = pl.kernel(
        _body,
        out_type=jax.ShapeDtypeStruct((B * D,), jnp.float32),
        mesh=mesh,
        scratch_types=[
            pltpu.VMEM((BPW * L,), jnp.int32),
            pltpu.VMEM((2, ROWS, D), jnp.float32),
            pltpu.VMEM((2, CH * D), jnp.float32),
            pltpu.SemaphoreType.DMA,
            pltpu.SemaphoreType.DMA,
        ],
        compiler_params=pltpu.CompilerParams(use_tc_tiling_on_sc=False),
    )
    return f(idx_m, table_rows).reshape(B, D)


def kernel(indices, table):
    idx = indices.astype(jnp.int32).reshape(B * L)
    idx_m = ((((idx >> 8) << 7) | (idx & 127)) << 1) | ((idx >> 7) & 1)
    table_rows = _relayout(table.astype(jnp.float32).T).reshape(2 * VROWS, D)
    return _run(idx_m, table_rows)
